# speedup vs baseline: 1.5521x; 1.5521x over previous
import functools

import jax
import jax.numpy as jnp
from jax import lax
from jax.experimental import pallas as pl
from jax.experimental.pallas import tpu as pltpu
from jax.experimental.pallas import tpu_sc as plsc

B = 16384
S = 50
D = 128
NC = 2
NS = 16
NW = NC * NS
BPW = B // NW
C_ELEM = 4
ROWS = C_ELEM * S
NCHUNK = BPW // C_ELEM
LANES = 16
NJ = D // LANES


def _sc_pool_body(x_hbm, emb_hbm, out_hbm, idx_v, rows_v, out_v, sem):
    cid = lax.axis_index("c")
    sid = lax.axis_index("s")
    wid = sid * NC + cid
    ebase = wid * BPW
    ibase = ebase * S

    @pl.loop(0, NCHUNK)
    def _chunk(c):
        pltpu.sync_copy(x_hbm.at[pl.ds(ibase + c * ROWS, ROWS)], idx_v)
        pltpu.async_copy(emb_hbm.at[idx_v], rows_v, sem).wait()
        for e in range(C_ELEM):
            for j in range(NJ):
                acc = rows_v[e * S, pl.ds(j * LANES, LANES)]
                for r in range(1, S):
                    acc = acc + rows_v[e * S + r, pl.ds(j * LANES, LANES)]
                out_v[e, pl.ds(j * LANES, LANES)] = acc * (1.0 / S)
        pltpu.sync_copy(out_v, out_hbm.at[pl.ds(ebase + c * C_ELEM, C_ELEM)])


def _sc_pool(x_flat, emb):
    mesh = plsc.VectorSubcoreMesh(
        core_axis_name="c", subcore_axis_name="s", num_cores=NC, num_subcores=NS
    )
    return pl.kernel(
        _sc_pool_body,
        out_type=jax.ShapeDtypeStruct((B, D), jnp.float32),
        mesh=mesh,
        scratch_types=[
            pltpu.VMEM((ROWS,), jnp.int32),
            pltpu.VMEM((ROWS, D), jnp.float32),
            pltpu.VMEM((C_ELEM, D), jnp.float32),
            pltpu.SemaphoreType.DMA,
        ],
    )(x_flat, emb)


def _mlp_body(p_ref, W1_ref, b1_ref, W2_ref, b2_ref, o_ref):
    h = p_ref[...]
    h1 = jnp.dot(h, W1_ref[...], preferred_element_type=jnp.float32) + b1_ref[...]
    h1 = jnp.maximum(h1, 0.0)
    h2 = jnp.dot(h1, W2_ref[...], preferred_element_type=jnp.float32) + b2_ref[...]
    norm = jnp.sqrt(jnp.sum(h2 * h2, axis=-1, keepdims=True))
    o_ref[...] = h2 / jnp.maximum(norm, 1e-12)


def _mlp(pooled, W1, b1, W2, b2):
    BLK = 2048
    grid = B // BLK
    return pl.pallas_call(
        _mlp_body,
        grid=(grid,),
        in_specs=[
            pl.BlockSpec((BLK, D), lambda i: (i, 0)),
            pl.BlockSpec((D, 64), lambda i: (0, 0)),
            pl.BlockSpec((1, 64), lambda i: (0, 0)),
            pl.BlockSpec((64, 10), lambda i: (0, 0)),
            pl.BlockSpec((1, 10), lambda i: (0, 0)),
        ],
        out_specs=pl.BlockSpec((BLK, 10), lambda i: (i, 0)),
        out_shape=jax.ShapeDtypeStruct((B, 10), jnp.float32),
    )(pooled, W1, b1, W2, b2)


def kernel(x, emb, W1, b1, W2, b2):
    pooled = _sc_pool(x.reshape(-1), emb)
    return _mlp(pooled, W1, b1.reshape(1, 64), W2, b2.reshape(1, 10))
